# Initial kernel scaffold; baseline (speedup 1.0000x reference)
import jax
import jax.numpy as jnp
from jax.experimental import pallas as pl


def kernel(x, edge_index, W_lin1, b_lin1, Wg, Ws_self, Ws_nb, Wi, Wl, W_la, W_cls, b_cls, na_alphas, sc_alphas, la_alphas):
    raise NotImplementedError("write your pallas kernel here")


# speedup vs baseline: 3.9059x; 3.9059x over previous
import functools

import jax
import jax.numpy as jnp
from jax import lax
from jax.experimental import pallas as pl
from jax.experimental.pallas import tpu as pltpu
from jax.experimental.pallas import tpu_sc as plsc

N = 10000
D = 128
E = 320000
C = 10

NC = 2
NS = 16
L = 16

NPAD = 10240
ROWS_PER_SUB = NPAD // NS
K = 128
BATCH_PER_SUB = 160
EPAD = NS * BATCH_PER_SUB * K

RB = 1000
GRID = N // RB

_mesh = plsc.VectorSubcoreMesh(core_axis_name="c", subcore_axis_name="s")


@functools.partial(
    pl.kernel,
    out_type=(
        jax.ShapeDtypeStruct((NPAD, D), jnp.float32),
        jax.ShapeDtypeStruct((NPAD, D), jnp.float32),
    ),
    mesh=_mesh,
    scratch_types=[
        pltpu.VMEM((K,), jnp.int32),
        pltpu.VMEM((K, D), jnp.float32),
        pltpu.VMEM_SHARED((NPAD, D), jnp.float32),
    ],
)
def _degree_kernel(dstp_hbm, srcq_hbm, ones_hbm, zrows_hbm, d0, d1, idx,
                   onesbuf, acc):
    c = lax.axis_index("c")
    s = lax.axis_index("s")
    pltpu.sync_copy(ones_hbm, onesbuf)
    pltpu.sync_copy(zrows_hbm, acc.at[pl.ds(s * ROWS_PER_SUB, ROWS_PER_SUB), :])
    plsc.subcore_barrier()

    def run(ei, out):
        def go():
            base0 = s * (BATCH_PER_SUB * K)

            def step(b, carry):
                pltpu.sync_copy(ei.at[pl.ds(base0 + b * K, K)], idx)
                pltpu.sync_copy(onesbuf, acc.at[idx], add=True)
                return carry

            lax.fori_loop(0, BATCH_PER_SUB, step, 0)
            plsc.subcore_barrier()
            pltpu.sync_copy(
                acc.at[pl.ds(s * ROWS_PER_SUB, ROWS_PER_SUB), :],
                out.at[pl.ds(s * ROWS_PER_SUB, ROWS_PER_SUB), :],
            )
        return go

    pl.when(c == 0)(run(dstp_hbm, d0))
    pl.when(c == 1)(run(srcq_hbm, d1))


@functools.partial(
    pl.kernel,
    out_type=(
        jax.ShapeDtypeStruct((NPAD, D), jnp.float32),
        jax.ShapeDtypeStruct((NPAD, D), jnp.float32),
    ),
    mesh=_mesh,
    scratch_types=[
        pltpu.VMEM((K,), jnp.int32),
        pltpu.VMEM((K,), jnp.int32),
        pltpu.VMEM((K, D), jnp.float32),
        pltpu.VMEM_SHARED((NPAD, D), jnp.float32),
        pltpu.SemaphoreType.DMA,
    ],
)
def _segsum_kernel(u0, u1, srcp, dstp, zrows, z0, z1, sidx, didx, rowbuf, acc, sem):
    c = lax.axis_index("c")
    s = lax.axis_index("s")
    pltpu.sync_copy(zrows, acc.at[pl.ds(s * ROWS_PER_SUB, ROWS_PER_SUB), :])
    plsc.subcore_barrier()

    def run(u, z):
        def go():
            base0 = s * (BATCH_PER_SUB * K)

            def step(b, carry):
                base = base0 + b * K
                pltpu.sync_copy(srcp.at[pl.ds(base, K)], sidx)
                pltpu.sync_copy(dstp.at[pl.ds(base, K)], didx)
                pltpu.async_copy(u.at[sidx], rowbuf, sem).wait()
                pltpu.sync_copy(rowbuf, acc.at[didx], add=True)
                return carry

            lax.fori_loop(0, BATCH_PER_SUB, step, 0)
            plsc.subcore_barrier()
            pltpu.sync_copy(
                acc.at[pl.ds(s * ROWS_PER_SUB, ROWS_PER_SUB), :],
                z.at[pl.ds(s * ROWS_PER_SUB, ROWS_PER_SUB), :],
            )
        return go

    pl.when(c == 0)(run(u0, z0))
    pl.when(c == 1)(run(u1, z1))


def _dotf(a, b):
    return jnp.dot(a, b, preferred_element_type=jnp.float32)


def _elu(v):
    return jnp.where(v > 0, v, jnp.exp(v) - 1.0)


def _lin1_body(x_ref, w_ref, b_ref, o_ref):
    o_ref[...] = _dotf(x_ref[...], w_ref[...]) + b_ref[...]


_lin1 = pl.pallas_call(
    _lin1_body,
    grid=(GRID,),
    in_specs=[
        pl.BlockSpec((RB, D), lambda i: (i, 0)),
        pl.BlockSpec((D, D), lambda i: (0, 0)),
        pl.BlockSpec((1, D), lambda i: (0, 0)),
    ],
    out_specs=pl.BlockSpec((RB, D), lambda i: (i, 0)),
    out_shape=jax.ShapeDtypeStruct((N, D), jnp.float32),
)


def _pre_body(x_ref, w_ref, degt_ref, o_ref):
    a = lax.rsqrt(jnp.maximum(degt_ref[:, 1:2], 1.0))
    o_ref[...] = _dotf(x_ref[...], w_ref[...]) * a


_pre = pl.pallas_call(
    _pre_body,
    grid=(GRID,),
    in_specs=[
        pl.BlockSpec((RB, D), lambda i: (i, 0)),
        pl.BlockSpec((D, D), lambda i: (0, 0)),
        pl.BlockSpec((RB, 2), lambda i: (i, 0)),
    ],
    out_specs=pl.BlockSpec((RB, D), lambda i: (i, 0)),
    out_shape=jax.ShapeDtypeStruct((N, D), jnp.float32),
)


def _post_body(x_ref, zs_ref, zg_ref, degt_ref, wself_ref, wnb_ref, wi_ref,
               wl_ref, wna_ref, o_ref):
    deg_in = jnp.maximum(degt_ref[:, 0:1], 1.0)
    x = x_ref[...]
    zs = zs_ref[...]
    op0 = zg_ref[...] * lax.rsqrt(deg_in)
    op1 = _dotf(x, wself_ref[...]) + _dotf(zs / deg_in, wnb_ref[...])
    op2 = _dotf(x + zs, wi_ref[...])
    op3 = _dotf(x, wl_ref[...])
    w = wna_ref
    o_ref[...] = (w[0, 0] * _elu(op0) + w[0, 1] * _elu(op1)
                  + w[0, 2] * _elu(op2) + w[0, 3] * _elu(op3))


_post = pl.pallas_call(
    _post_body,
    grid=(GRID,),
    in_specs=[
        pl.BlockSpec((RB, D), lambda i: (i, 0)),
        pl.BlockSpec((RB, D), lambda i: (i, 0)),
        pl.BlockSpec((RB, D), lambda i: (i, 0)),
        pl.BlockSpec((RB, 2), lambda i: (i, 0)),
        pl.BlockSpec((D, D), lambda i: (0, 0)),
        pl.BlockSpec((D, D), lambda i: (0, 0)),
        pl.BlockSpec((D, D), lambda i: (0, 0)),
        pl.BlockSpec((D, D), lambda i: (0, 0)),
        pl.BlockSpec((1, 4), lambda i: (0, 0)),
    ],
    out_specs=pl.BlockSpec((RB, D), lambda i: (i, 0)),
    out_shape=jax.ShapeDtypeStruct((N, D), jnp.float32),
)


def _final_body(x1_ref, x2_ref, x3_ref, wla_ref, wcls_ref, bcls_ref, q_ref, o_ref):
    q = q_ref
    s1 = q[0, 0] * x1_ref[...]
    s2 = q[0, 1] * x2_ref[...]
    x3 = x3_ref[...]
    op0 = s1 + s2 + x3
    op1 = jnp.maximum(jnp.maximum(s1, s2), x3)
    op2 = (_dotf(s1, wla_ref[0:D, :]) + _dotf(s2, wla_ref[D:2 * D, :])
           + _dotf(x3, wla_ref[2 * D:3 * D, :]))
    x5 = q[0, 2] * _elu(op0) + q[0, 3] * _elu(op1) + q[0, 4] * _elu(op2)
    o_ref[...] = _dotf(x5, wcls_ref[...]) + bcls_ref[...]


_final = pl.pallas_call(
    _final_body,
    grid=(GRID,),
    in_specs=[
        pl.BlockSpec((RB, D), lambda i: (i, 0)),
        pl.BlockSpec((RB, D), lambda i: (i, 0)),
        pl.BlockSpec((RB, D), lambda i: (i, 0)),
        pl.BlockSpec((3 * D, D), lambda i: (0, 0)),
        pl.BlockSpec((D, C), lambda i: (0, 0)),
        pl.BlockSpec((1, C), lambda i: (0, 0)),
        pl.BlockSpec((1, 5), lambda i: (0, 0)),
    ],
    out_specs=pl.BlockSpec((RB, C), lambda i: (i, 0)),
    out_shape=jax.ShapeDtypeStruct((N, C), jnp.float32),
)


def kernel(x, edge_index, W_lin1, b_lin1, Wg, Ws_self, Ws_nb, Wi, Wl, W_la,
           W_cls, b_cls, na_alphas, sc_alphas, la_alphas):
    src = edge_index[0]
    dst = edge_index[1]
    pad = jnp.full((EPAD - E,), NPAD - 1, jnp.int32)
    srcp = jnp.concatenate([src, jnp.zeros((EPAD - E,), jnp.int32)])
    srcq = jnp.concatenate([src, pad])
    dstp = jnp.concatenate([dst, pad])
    ones_rows = jnp.ones((K, D), jnp.float32)
    zrows = jnp.zeros((ROWS_PER_SUB, D), jnp.float32)

    d0, d1 = _degree_kernel(dstp, srcq, ones_rows, zrows)
    degt = jnp.stack([d0[:, 0], d1[:, 0]], axis=1)

    wna = jax.nn.softmax(na_alphas, axis=-1)
    q1 = jax.nn.softmax(sc_alphas[0])[0]
    q2 = jax.nn.softmax(sc_alphas[1])[0]
    wla = jax.nn.softmax(la_alphas[0])
    scal = jnp.stack([q1, q2, wla[0], wla[1], wla[2]]).reshape(1, 5)

    xcur = _lin1(x, W_lin1, b_lin1.reshape(1, D))
    xs = []
    for l in range(3):
        u1 = _pre(xcur, Wg[l], degt)
        z0, z1 = _segsum_kernel(xcur, u1, srcp, dstp, zrows)
        xcur = _post(xcur, z0, z1, degt, Ws_self[l], Ws_nb[l], Wi[l], Wl[l],
                     wna[l].reshape(1, 4))
        xs.append(xcur)

    return _final(xs[0], xs[1], xs[2], W_la, W_cls, b_cls.reshape(1, C), scal)
